# Initial kernel scaffold; baseline (speedup 1.0000x reference)
import jax
import jax.numpy as jnp
from jax.experimental import pallas as pl


def kernel(action, explore_rate, step, obs):
    raise NotImplementedError("write your pallas kernel here")


# speedup vs baseline: 1.2645x; 1.2645x over previous
import jax
import jax.numpy as jnp
from jax.experimental import pallas as pl

B, E, A = 4096, 8, 1000
BLOCK_B = 128


def _body(a_ref, e_ref, o_ref):
    best = a_ref[:, 0, :]
    bstd = e_ref[:, 0, :]
    for k in range(1, E):
        ak = a_ref[:, k, :]
        ek = e_ref[:, k, :]
        take = ak < best
        bstd = jnp.where(take, ek, bstd)
        best = jnp.where(take, ak, best)
    scaled = best / jnp.maximum(bstd, 1e-8)
    m = jnp.max(scaled, axis=-1, keepdims=True)
    p = jnp.exp(scaled - m)
    o_ref[...] = p / jnp.sum(p, axis=-1, keepdims=True)


def kernel(action, explore_rate, step, obs):
    del step, obs
    return pl.pallas_call(
        _body,
        grid=(B // BLOCK_B,),
        in_specs=[
            pl.BlockSpec((BLOCK_B, E, A), lambda i: (i, 0, 0)),
            pl.BlockSpec((BLOCK_B, E, A), lambda i: (i, 0, 0)),
        ],
        out_specs=pl.BlockSpec((BLOCK_B, A), lambda i: (i, 0)),
        out_shape=jax.ShapeDtypeStruct((B, A), jnp.float32),
    )(action, explore_rate)
